# baseline (speedup 1.0000x reference)
import functools

import jax
import jax.numpy as jnp
from jax import lax
from jax.experimental import pallas as pl
from jax.experimental.pallas import tpu as pltpu
from jax.experimental.pallas import tpu_sc as plsc

DIM = 16
DP = DIM // 2
BATCH = 4096
NROWS = 1000000
PADCOLS = 1 << 20
NBLK = 16
CHUNK = PADCOLS // NBLK
NC = 2
NS = 16
NW = NC * NS
UPW = BATCH // NW
IPS = BATCH // NS


def _repack_body(in_ref, out_ref):
    a = in_ref[...].reshape(DP, 2, CHUNK)
    lo = lax.bitcast_convert_type(
        a[:, 0, :].astype(jnp.bfloat16), jnp.uint16).astype(jnp.uint32)
    hi = lax.bitcast_convert_type(
        a[:, 1, :].astype(jnp.bfloat16), jnp.uint16).astype(jnp.uint32)
    packed = lax.bitcast_convert_type(lo | (hi << 16), jnp.float32)
    out_ref[...] = packed.reshape(DP, CHUNK // 128, 128)


def _repack(table_t):
    return pl.pallas_call(
        _repack_body,
        grid=((NROWS + CHUNK - 1) // CHUNK,),
        in_specs=[pl.BlockSpec((DIM, CHUNK), lambda i: (0, i))],
        out_specs=pl.BlockSpec((DP, CHUNK // 128, 128), lambda i: (0, i, 0)),
        out_shape=jax.ShapeDtypeStruct((DP, PADCOLS // 128, 128),
                                       jnp.float32),
    )(table_t)


def _unpack(w):
    return plsc.unpack(plsc.bitcast(w, jnp.bfloat16),
                       format=plsc.PackFormat.INTERLEAVED)


@functools.partial(
    pl.kernel,
    mesh=plsc.VectorSubcoreMesh(core_axis_name="c", subcore_axis_name="s"),
    out_type=jax.ShapeDtypeStruct((BATCH,), jnp.float32),
    compiler_params=pltpu.CompilerParams(
        needs_layout_passes=False, use_tc_tiling_on_sc=False),
    scratch_types=[
        pltpu.VMEM((DP * UPW,), jnp.int32),
        pltpu.VMEM((DP * IPS,), jnp.int32),
        pltpu.VMEM((DP * UPW,), jnp.float32),
        pltpu.VMEM((DP * IPS,), jnp.float32),
        pltpu.VMEM((128,), jnp.float32),
        pltpu.VMEM((NS, 128), jnp.float32),
        pltpu.VMEM((UPW,), jnp.float32),
        pltpu.VMEM((16, 16), jnp.float32),
        pltpu.VMEM_SHARED((NS, 128), jnp.float32),
        pltpu.SemaphoreType.DMA,
        pltpu.SemaphoreType.DMA,
    ],
)
def _mf_kernel(uflat_hbm, iflat_hbm, u_hbm, v_hbm, out_hbm,
               uidx, iidx, uel, iel, part, allparts, outv, tile, shared,
               sem_u, sem_i):
    c = lax.axis_index("c")
    s = lax.axis_index("s")
    wid = s * NC + c
    ubase = wid * UPW

    pltpu.sync_copy(uflat_hbm.at[pl.ds(wid * DP * UPW, DP * UPW)], uidx)
    pltpu.sync_copy(iflat_hbm.at[pl.ds(s * DP * IPS, DP * IPS)], iidx)

    ucps = [
        pltpu.async_copy(u_hbm.at[uidx.at[pl.ds(d * UPW, UPW)]],
                         uel.at[pl.ds(d * UPW, UPW)], sem_u)
        for d in range(DP)
    ]
    icps = [
        pltpu.async_copy(v_hbm.at[iidx.at[pl.ds(d * IPS + j * 128, 128)]],
                         iel.at[pl.ds(d * IPS + j * 128, 128)], sem_i)
        for d in range(DP) for j in range(IPS // 128)
    ]
    for cp in icps:
        cp.wait()

    lane = lax.iota(jnp.int32, 16)
    accs = []
    for d in range(DP):
        alo, ahi = _unpack(iel[pl.ds(d * IPS, 16)])
        for k in range(1, IPS // 16):
            lo, hi = _unpack(iel[pl.ds(d * IPS + k * 16, 16)])
            alo = alo + lo
            ahi = ahi + hi
        accs.extend((alo, ahi))

    for d in range(DIM):
        plsc.store_scatter(tile, [lane, jnp.full((16,), d, jnp.int32)],
                           accs[d])
    acc = tile[0, :]
    for r in range(1, 16):
        acc = acc + tile[r, :]
    part[pl.ds(0, DIM)] = acc

    pltpu.sync_copy(part, shared.at[s])
    plsc.subcore_barrier()
    pltpu.sync_copy(shared, allparts)
    svec = jnp.zeros((DIM,), jnp.float32)
    for t in range(NS):
        svec = svec + allparts[t, pl.ds(0, DIM)]

    sd = [jnp.take_along_axis(svec, jnp.full((16,), d, jnp.int32), axis=0)
          for d in range(DIM)]

    for cp in ucps:
        cp.wait()

    def dot_body(g, carry):
        ovec = jnp.zeros((16,), jnp.float32)
        for d in range(DP):
            lo, hi = _unpack(uel[pl.ds(d * UPW + g * 16, 16)])
            ovec = ovec + sd[2 * d] * lo + sd[2 * d + 1] * hi
        outv[pl.ds(g * 16, 16)] = ovec
        return carry
    lax.fori_loop(0, UPW // 16, dot_body, jnp.int32(0))

    pltpu.sync_copy(outv, out_hbm.at[pl.ds(ubase, UPW)])


def kernel(users, items, U, V):
    users = users.astype(jnp.int32)
    items = items.astype(jnp.int32)
    uf = _repack(U.T).reshape(-1)
    vf = _repack(V.T).reshape(-1)
    dofs = jnp.arange(DP, dtype=jnp.int32)[:, None] * PADCOLS
    uflat = (users[None, :] + dofs)
    uflat = uflat.reshape(DP, NW, UPW).transpose(1, 0, 2).reshape(-1)
    iflat = (items[None, :] + dofs)
    iflat = iflat.reshape(DP, NS, IPS).transpose(1, 0, 2).reshape(-1)
    return _mf_kernel(uflat, iflat, uf, vf)


# speedup vs baseline: 2.7133x; 2.7133x over previous
import functools

import jax
import jax.numpy as jnp
from jax import lax
from jax.experimental import pallas as pl
from jax.experimental.pallas import tpu as pltpu
from jax.experimental.pallas import tpu_sc as plsc

DIM = 16
BATCH = 4096
NROWS = 1000000
PADCOLS = 1 << 20
NBLK = 8
CHUNK = PADCOLS // NBLK
NC = 2
NS = 16
NW = NC * NS
UPW = BATCH // NW
IPS = BATCH // NS


def _depad_body(in_ref, out_ref):
    out_ref[...] = in_ref[...].reshape(DIM, CHUNK // 128, 128)


def _depad(table_t):
    return pl.pallas_call(
        _depad_body,
        grid=((NROWS + CHUNK - 1) // CHUNK,),
        compiler_params=pltpu.CompilerParams(
            vmem_limit_bytes=100 * 1024 * 1024),
        in_specs=[pl.BlockSpec((DIM, CHUNK), lambda i: (0, i))],
        out_specs=pl.BlockSpec((DIM, CHUNK // 128, 128), lambda i: (0, i, 0)),
        out_shape=jax.ShapeDtypeStruct((DIM, PADCOLS // 128, 128),
                                       jnp.float32),
    )(table_t)


@functools.partial(
    pl.kernel,
    mesh=plsc.VectorSubcoreMesh(core_axis_name="c", subcore_axis_name="s"),
    out_type=jax.ShapeDtypeStruct((BATCH,), jnp.float32),
    compiler_params=pltpu.CompilerParams(
        needs_layout_passes=False, use_tc_tiling_on_sc=False),
    scratch_types=[
        pltpu.VMEM((DIM * UPW,), jnp.int32),
        pltpu.VMEM((DIM * IPS,), jnp.int32),
        pltpu.VMEM((DIM * UPW,), jnp.float32),
        pltpu.VMEM((DIM * IPS,), jnp.float32),
        pltpu.VMEM((128,), jnp.float32),
        pltpu.VMEM((NS, 128), jnp.float32),
        pltpu.VMEM((UPW,), jnp.float32),
        pltpu.VMEM((16, 16), jnp.float32),
        pltpu.VMEM_SHARED((NS, 128), jnp.float32),
        pltpu.SemaphoreType.DMA,
        pltpu.SemaphoreType.DMA,
    ],
)
def _mf_kernel(uflat_hbm, iflat_hbm, u_hbm, v_hbm, out_hbm,
               uidx, iidx, uel, iel, part, allparts, outv, tile, shared,
               sem_u, sem_i):
    c = lax.axis_index("c")
    s = lax.axis_index("s")
    wid = s * NC + c
    ubase = wid * UPW

    pltpu.sync_copy(uflat_hbm.at[pl.ds(wid * DIM * UPW, DIM * UPW)], uidx)
    pltpu.sync_copy(iflat_hbm.at[pl.ds(s * DIM * IPS, DIM * IPS)], iidx)

    ucps = [
        pltpu.async_copy(u_hbm.at[uidx.at[pl.ds(d * UPW, UPW)]],
                         uel.at[pl.ds(d * UPW, UPW)], sem_u)
        for d in range(DIM)
    ]
    icps = [
        pltpu.async_copy(v_hbm.at[iidx.at[pl.ds(d * IPS + j * 128, 128)]],
                         iel.at[pl.ds(d * IPS + j * 128, 128)], sem_i)
        for d in range(DIM) for j in range(IPS // 128)
    ]
    for cp in icps:
        cp.wait()

    lane = lax.iota(jnp.int32, 16)
    accs = []
    for d in range(DIM):
        a = iel[pl.ds(d * IPS, 16)]
        for k in range(1, IPS // 16):
            a = a + iel[pl.ds(d * IPS + k * 16, 16)]
        accs.append(a)

    for d in range(DIM):
        plsc.store_scatter(tile, [lane, jnp.full((16,), d, jnp.int32)],
                           accs[d])
    acc = tile[0, :]
    for r in range(1, 16):
        acc = acc + tile[r, :]
    part[pl.ds(0, DIM)] = acc

    pltpu.sync_copy(part, shared.at[s])
    plsc.subcore_barrier()
    pltpu.sync_copy(shared, allparts)
    svec = jnp.zeros((DIM,), jnp.float32)
    for t in range(NS):
        svec = svec + allparts[t, pl.ds(0, DIM)]

    sd = [jnp.take_along_axis(svec, jnp.full((16,), d, jnp.int32), axis=0)
          for d in range(DIM)]

    for cp in ucps:
        cp.wait()

    def dot_body(g, carry):
        ovec = sd[0] * uel[pl.ds(g * 16, 16)]
        for d in range(1, DIM):
            ovec = ovec + sd[d] * uel[pl.ds(d * UPW + g * 16, 16)]
        outv[pl.ds(g * 16, 16)] = ovec
        return carry
    lax.fori_loop(0, UPW // 16, dot_body, jnp.int32(0))

    pltpu.sync_copy(outv, out_hbm.at[pl.ds(ubase, UPW)])


def kernel(users, items, U, V):
    users = users.astype(jnp.int32)
    items = items.astype(jnp.int32)
    uf = _depad(U.T).reshape(-1)
    vf = _depad(V.T).reshape(-1)
    dofs = jnp.arange(DIM, dtype=jnp.int32)[:, None] * PADCOLS
    uflat = (users[None, :] + dofs)
    uflat = uflat.reshape(DIM, NW, UPW).transpose(1, 0, 2).reshape(-1)
    iflat = (items[None, :] + dofs)
    iflat = iflat.reshape(DIM, NS, IPS).transpose(1, 0, 2).reshape(-1)
    return _mf_kernel(uflat, iflat, uf, vf)
